# baseline (speedup 1.0000x reference)
import functools

import jax
import jax.numpy as jnp
from jax import lax
from jax.experimental import pallas as pl
from jax.experimental.pallas import tpu as pltpu
from jax.experimental.pallas import tpu_sc as plsc

_NC = 2
_NS = 16
_NW = _NC * _NS
_L = 16

_CH = 128


def _sc_trilinear(table, xs, ys, zs, params, *, n_pts, sizes, c_pad):
  per_tile = n_pts // _NW
  n_chunks = per_tile // _CH
  sx, sy, sz = sizes
  stride_x = sy * sz
  stride_y = sz

  mesh = plsc.VectorSubcoreMesh(core_axis_name="c", subcore_axis_name="s")

  @functools.partial(
      pl.kernel,
      out_type=jax.ShapeDtypeStruct((n_pts, c_pad), jnp.float32),
      mesh=mesh,
      compiler_params=pltpu.CompilerParams(use_tc_tiling_on_sc=False),
      scratch_types=[
          pltpu.VMEM((_L,), jnp.float32),
          pltpu.VMEM((_CH,), jnp.float32),
          pltpu.VMEM((_CH,), jnp.float32),
          pltpu.VMEM((_CH,), jnp.float32),
          pltpu.VMEM((8, _CH), jnp.int32),
          pltpu.VMEM((8, _CH), jnp.float32),
          pltpu.VMEM((8, _CH, c_pad), jnp.float32),
          pltpu.VMEM((_CH, c_pad), jnp.float32),
          pltpu.SemaphoreType.DMA,
      ],
  )
  def k(table_h, xs_h, ys_h, zs_h, params_h, out_h,
        params_v, x_v, y_v, z_v, idx_b, w_b, rows_b, out_b, sem):
    wid = lax.axis_index("s") * _NC + lax.axis_index("c")
    base0 = wid * per_tile

    pltpu.sync_copy(params_h, params_v)
    pv = params_v[...]
    mn0 = pv[0]
    mn1 = pv[1]
    mn2 = pv[2]
    s0 = pv[3]
    s1 = pv[4]
    s2 = pv[5]

    def chunk_body(t, _):
      base = pl.multiple_of(base0 + t * _CH, _CH)
      pltpu.sync_copy(xs_h.at[pl.ds(base, _CH)], x_v)
      pltpu.sync_copy(ys_h.at[pl.ds(base, _CH)], y_v)
      pltpu.sync_copy(zs_h.at[pl.ds(base, _CH)], z_v)

      for j in range(_CH // _L):
        sl = pl.ds(j * _L, _L)
        ux = (x_v[sl] - mn0) * s0
        uy = (y_v[sl] - mn1) * s1
        uz = (z_v[sl] - mn2) * s2
        ix0 = jnp.clip(ux.astype(jnp.int32), 0, sx - 2)
        iy0 = jnp.clip(uy.astype(jnp.int32), 0, sy - 2)
        iz0 = jnp.clip(uz.astype(jnp.int32), 0, sz - 2)
        fx = ux - ix0.astype(jnp.float32)
        fy = uy - iy0.astype(jnp.float32)
        fz = uz - iz0.astype(jnp.float32)
        gx = 1.0 - fx
        gy = 1.0 - fy
        gz = 1.0 - fz
        ax0 = ix0 * stride_x
        ax1 = ax0 + stride_x
        by0 = iy0 * stride_y
        by1 = by0 + stride_y
        a00 = ax0 + by0 + iz0
        a01 = ax0 + by1 + iz0
        a10 = ax1 + by0 + iz0
        a11 = ax1 + by1 + iz0
        idx_b[0, sl] = a00
        idx_b[1, sl] = a00 + 1
        idx_b[2, sl] = a01
        idx_b[3, sl] = a01 + 1
        idx_b[4, sl] = a10
        idx_b[5, sl] = a10 + 1
        idx_b[6, sl] = a11
        idx_b[7, sl] = a11 + 1
        wxy00 = gx * gy
        wxy01 = gx * fy
        wxy10 = fx * gy
        wxy11 = fx * fy
        w_b[0, sl] = wxy00 * gz
        w_b[1, sl] = wxy00 * fz
        w_b[2, sl] = wxy01 * gz
        w_b[3, sl] = wxy01 * fz
        w_b[4, sl] = wxy10 * gz
        w_b[5, sl] = wxy10 * fz
        w_b[6, sl] = wxy11 * gz
        w_b[7, sl] = wxy11 * fz

      descs = [
          pltpu.async_copy(table_h.at[idx_b.at[c]], rows_b.at[c], sem)
          for c in range(8)
      ]
      for d in descs:
        d.wait()

      def acc_group(g, _):
        gbase = g * _L
        wvs = [w_b[c, pl.ds(gbase, _L)] for c in range(8)]
        for p in range(_L):
          i = gbase + p
          acc = rows_b[0, i] * wvs[0][p]
          for c in range(1, 8):
            acc = acc + rows_b[c, i] * wvs[c][p]
          out_b[i] = acc
        return 0

      lax.fori_loop(0, _CH // _L, acc_group, 0)

      pltpu.sync_copy(out_b, out_h.at[pl.ds(base, _CH)])
      return 0

    lax.fori_loop(0, n_chunks, chunk_body, 0)

  return k(table, xs, ys, zs, params)


def kernel(xyz, grid, xyz_min, xyz_max):
  c, sx, sy, sz = grid.shape
  n_pts = xyz.shape[0]
  c_pad = 16
  table = jnp.pad(grid, ((0, c_pad - c), (0, 0), (0, 0), (0, 0)))
  table = table.transpose(1, 2, 3, 0).reshape(-1, c_pad)
  xs = xyz[:, 0]
  ys = xyz[:, 1]
  zs = xyz[:, 2]
  sizes_f = jnp.array([sx - 1, sy - 1, sz - 1], jnp.float32)
  scale = sizes_f / (xyz_max.astype(jnp.float32) - xyz_min.astype(jnp.float32))
  params = jnp.concatenate(
      [xyz_min.astype(jnp.float32), scale, jnp.zeros((10,), jnp.float32)])
  out = _sc_trilinear(table, xs, ys, zs, params,
                      n_pts=n_pts, sizes=(int(sx), int(sy), int(sz)),
                      c_pad=c_pad)
  return out[:, :c]
